# Initial kernel scaffold; baseline (speedup 1.0000x reference)
import jax
import jax.numpy as jnp
from jax.experimental import pallas as pl


def kernel(energy, word_h, e1, e2, sent_len, rel_embs, arc_W, arc_b, kv_W, kv_b, hid_W, hid_b):
    raise NotImplementedError("write your pallas kernel here")


# speedup vs baseline: 3.2742x; 3.2742x over previous
import jax
import jax.numpy as jnp
from jax.experimental import pallas as pl
from jax.experimental.pallas import tpu as pltpu

_L = 128
_R = 40
_D2 = 512
_D4 = 1024
_EMB = 15
_LL = _L * _L
_NBLK = 8
_COLS = _LL // _NBLK
_IPB = _COLS // _L
_CHUNK = 2048

_HIGH = jax.lax.Precision.HIGHEST
_ANY_SPEC = pl.BlockSpec(memory_space=pltpu.MemorySpace.HBM)


def _leaky(x):
    return jnp.where(x >= 0, x, 0.01 * x)


def _dotT(a, b, precision):
    return jax.lax.dot_general(
        a, b, (((1,), (1,)), ((), ())),
        precision=precision, preferred_element_type=jnp.float32)


def _phase0_body(wr_ref, wh_ref, wd_ref, wrp_ref, rel_ref,
                 at_ref, bt_ref, ct_ref):
    at_ref[:] = _dotT(wh_ref[:], wr_ref[:], _HIGH)
    bt_ref[:] = _dotT(wd_ref[:], wr_ref[:], _HIGH)
    ct_ref[:] = _dotT(wrp_ref[:], rel_ref[:], _HIGH)


def _phase1_body(e_ref, at3_ref, bt_ref, ct_ref, ab_ref, mbT_ref, marg_ref):
    e_blk = e_ref[:]
    relcT = jax.lax.dot_general(
        ct_ref[:], e_blk, (((1,), (0,)), ((), ())),
        preferred_element_type=jnp.float32)
    w_row = jnp.sum(e_blk, axis=0, keepdims=True)
    marg_ref[:] = w_row
    ab = ab_ref[:]
    bt = bt_ref[:]
    for t in range(_IPB):
        a_col = at3_ref[0, :, t:t + 1]
        sl = slice(t * _L, (t + 1) * _L)
        z = w_row[:, sl] * (a_col + bt) + relcT[:, sl] + ab
        mbT_ref[:, sl] = _leaky(z)


def _phase2_body(mbT_ref, marg_ref, sl_ref, e12_ref, kvw_hbm, kvbt_ref,
                 hidwt_hbm, hidb_ref, out_ref, kv_scr, hid_scr, sem1, sem2):
    marg = marg_ref[:]
    sl = sl_ref[0, 0]
    pidx = jax.lax.broadcasted_iota(jnp.int32, (1, _LL), 1)
    valid = ((pidx // _L) < sl) & ((pidx % _L) < sl)
    t0 = jnp.float32(1e-06)
    t1 = jnp.float32(1e-06 * 0.1)
    t2 = jnp.float32(1e-06 * 0.01)
    c0 = jnp.sum(((marg > t0) & valid).astype(jnp.float32))
    c1 = jnp.sum(((marg > t1) & valid).astype(jnp.float32))
    thr = jnp.where(c0 > 0, t0, jnp.where(c1 > 0, t1, t2))
    status = (marg > thr) & valid

    x = e12_ref[:]
    for i in range(3):
        cp1 = pltpu.make_async_copy(kvw_hbm.at[i], kv_scr, sem1)
        cp1.start()
        cp2 = pltpu.make_async_copy(hidwt_hbm.at[i], hid_scr, sem2)
        cp2.start()
        cp1.wait()
        kv_col = jnp.concatenate(
            [jnp.sum(kv_scr[h * _D2:(h + 1) * _D2, :] * x,
                     axis=1, keepdims=True) for h in range(2)],
            axis=0) + kvbt_ref[:, i:i + 1]
        key_col = jnp.tanh(kv_col[:_D2])
        val_col = _leaky(kv_col[_D2:])
        logits = jnp.concatenate(
            [jnp.sum(mbT_ref[:, c * _CHUNK:(c + 1) * _CHUNK] * key_col,
                     axis=0, keepdims=True)
             for c in range(_LL // _CHUNK)], axis=1)
        lm = jnp.where(status, logits, jnp.float32(-1e30))
        m = jnp.max(lm)
        ex = jnp.where(status, jnp.exp(logits - m), jnp.float32(0.0))
        s = jnp.sum(ex)
        dist = ex / s
        mem_rep = jnp.sum(mbT_ref[:, 0:_CHUNK] * dist[:, 0:_CHUNK],
                          axis=1, keepdims=True)
        for c in range(1, _LL // _CHUNK):
            csl = slice(c * _CHUNK, (c + 1) * _CHUNK)
            mem_rep = mem_rep + jnp.sum(mbT_ref[:, csl] * dist[:, csl],
                                        axis=1, keepdims=True)
        cat_col = jnp.concatenate([val_col, mem_rep], axis=0)
        cp2.wait()
        h_row = jnp.sum(hid_scr[0:_D2, :] * cat_col[0:_D2], axis=0,
                        keepdims=True)
        h_row = h_row + jnp.sum(hid_scr[_D2:, :] * cat_col[_D2:], axis=0,
                                keepdims=True)
        x = _leaky(h_row + hidb_ref[i:i + 1, :])
    out_ref[:] = x


def kernel(energy, word_h, e1, e2, sent_len, rel_embs, arc_W, arc_b,
           kv_W, kv_b, hid_W, hid_b):
    e2d = energy.reshape(_R, _LL)
    wr = word_h[0]
    wh_p = arc_W[:, :_D2]
    wrp_p = arc_W[:, _D2:_D2 + _EMB]
    wd_p = arc_W[:, _D2 + _EMB:]
    ab_col = arc_b.reshape(_D2, 1)

    at, bt, ct = pl.pallas_call(
        _phase0_body,
        out_shape=[
            jax.ShapeDtypeStruct((_D2, _L), jnp.float32),
            jax.ShapeDtypeStruct((_D2, _L), jnp.float32),
            jax.ShapeDtypeStruct((_D2, _R), jnp.float32),
        ],
    )(wr, wh_p, wd_p, wrp_p, rel_embs)

    at3 = jnp.pad(
        at.reshape(_D2, _NBLK, _IPB).transpose(1, 0, 2),
        ((0, 0), (0, 0), (0, _L - _IPB)))

    mbT, marg = pl.pallas_call(
        _phase1_body,
        grid=(_NBLK,),
        in_specs=[
            pl.BlockSpec((_R, _COLS), lambda b: (0, b)),
            pl.BlockSpec((1, _D2, _L), lambda b: (b, 0, 0)),
            pl.BlockSpec((_D2, _L), lambda b: (0, 0)),
            pl.BlockSpec((_D2, _R), lambda b: (0, 0)),
            pl.BlockSpec((_D2, 1), lambda b: (0, 0)),
        ],
        out_specs=[
            pl.BlockSpec((_D2, _COLS), lambda b: (0, b)),
            pl.BlockSpec((1, _COLS), lambda b: (0, b)),
        ],
        out_shape=[
            jax.ShapeDtypeStruct((_D2, _LL), jnp.float32),
            jax.ShapeDtypeStruct((1, _LL), jnp.float32),
        ],
    )(e2d, at3, bt, ct, ab_col)

    sl = jnp.asarray(sent_len, jnp.int32).reshape(1, 1)
    e12 = jnp.concatenate([e1, e2], axis=0).reshape(1, _D4)
    kvbt = kv_b.T
    hidwt = hid_W.transpose(0, 2, 1)

    out = pl.pallas_call(
        _phase2_body,
        in_specs=[
            _ANY_SPEC if idx in (4, 6) else pl.BlockSpec()
            for idx in range(8)
        ],
        out_shape=jax.ShapeDtypeStruct((1, _D4), jnp.float32),
        scratch_shapes=[
            pltpu.VMEM((_D4, _D4), jnp.float32),
            pltpu.VMEM((_D4, _D4), jnp.float32),
            pltpu.SemaphoreType.DMA,
            pltpu.SemaphoreType.DMA,
        ],
    )(mbT, marg, sl, e12, kv_W, kvbt, hidwt, hid_b)
    return out.reshape(_D4)


# speedup vs baseline: 4.0431x; 1.2348x over previous
import jax
import jax.numpy as jnp
from jax.experimental import pallas as pl
from jax.experimental.pallas import tpu as pltpu

_L = 128
_R = 40
_D2 = 512
_D4 = 1024
_EMB = 15
_LL = _L * _L
_NBLK = 8
_COLS = _LL // _NBLK
_IPB = _COLS // _L
_CHUNK = 2048

_HIGH = jax.lax.Precision.HIGHEST
_ANY_SPEC = pl.BlockSpec(memory_space=pltpu.MemorySpace.HBM)


def _leaky(x):
    return jnp.where(x >= 0, x, 0.01 * x)


def _dotT(a, b, precision):
    return jax.lax.dot_general(
        a, b, (((1,), (1,)), ((), ())),
        precision=precision, preferred_element_type=jnp.float32)


def _phase0_body(wr_ref, wh_ref, wd_ref, wrp_ref, rel_ref,
                 at_ref, bt_ref, ct_ref):
    at_ref[:] = _dotT(wh_ref[:], wr_ref[:], _HIGH)
    bt_ref[:] = _dotT(wd_ref[:], wr_ref[:], _HIGH)
    ct_ref[:] = _dotT(wrp_ref[:], rel_ref[:], _HIGH)


def _phase1_body(e_ref, at3_ref, bt_ref, ct_ref, ab_ref, mbT_ref, marg_ref):
    e_blk = e_ref[:]
    relcT = jax.lax.dot_general(
        ct_ref[:], e_blk, (((1,), (0,)), ((), ())),
        preferred_element_type=jnp.float32)
    w_row = jnp.sum(e_blk, axis=0, keepdims=True)
    marg_ref[:] = w_row
    ab = ab_ref[:]
    bt = bt_ref[:]
    for t in range(_IPB):
        a_col = at3_ref[0, :, t:t + 1]
        sl = slice(t * _L, (t + 1) * _L)
        z = w_row[:, sl] * (a_col + bt) + relcT[:, sl] + ab
        mbT_ref[:, sl] = _leaky(z)


def _phase2_body(mbT_ref, marg_ref, sl_ref, e12_ref, kvw_hbm, kvbt_ref,
                 hidwt_hbm, hidb_ref, out_ref, kv_scr, hid_scr, sem1, sem2):
    marg = marg_ref[:]
    sl = sl_ref[0, 0]
    pidx = jax.lax.broadcasted_iota(jnp.int32, (1, _LL), 1)
    valid = ((pidx // _L) < sl) & ((pidx % _L) < sl)
    t0 = jnp.float32(1e-06)
    t1 = jnp.float32(1e-06 * 0.1)
    t2 = jnp.float32(1e-06 * 0.01)
    c0 = jnp.sum(((marg > t0) & valid).astype(jnp.float32))
    c1 = jnp.sum(((marg > t1) & valid).astype(jnp.float32))
    thr = jnp.where(c0 > 0, t0, jnp.where(c1 > 0, t1, t2))
    status = (marg > thr) & valid

    x = e12_ref[:]
    for i in range(3):
        cp1 = pltpu.make_async_copy(kvw_hbm.at[i], kv_scr, sem1)
        cp1.start()
        cp2 = pltpu.make_async_copy(hidwt_hbm.at[i], hid_scr, sem2)
        cp2.start()
        cp1.wait()
        kv_col = jnp.concatenate(
            [jnp.sum(kv_scr[h * _D2:(h + 1) * _D2, :] * x,
                     axis=1, keepdims=True) for h in range(2)],
            axis=0) + kvbt_ref[:, i:i + 1]
        key_col = jnp.tanh(kv_col[:_D2])
        val_col = _leaky(kv_col[_D2:])
        logits = jnp.concatenate(
            [jnp.sum(mbT_ref[:, c * _CHUNK:(c + 1) * _CHUNK] * key_col,
                     axis=0, keepdims=True)
             for c in range(_LL // _CHUNK)], axis=1)
        lm = jnp.where(status, logits, jnp.float32(-1e30))
        m = jnp.max(lm)
        ex = jnp.where(status, jnp.exp(logits - m), jnp.float32(0.0))
        s = jnp.sum(ex)
        dist = ex / s
        mem_rep = jnp.sum(mbT_ref[:, 0:_CHUNK] * dist[:, 0:_CHUNK],
                          axis=1, keepdims=True)
        for c in range(1, _LL // _CHUNK):
            csl = slice(c * _CHUNK, (c + 1) * _CHUNK)
            mem_rep = mem_rep + jnp.sum(mbT_ref[:, csl] * dist[:, csl],
                                        axis=1, keepdims=True)
        cat_row = jnp.transpose(
            jnp.concatenate([val_col, mem_rep], axis=0))
        cp2.wait()
        h_col = jnp.sum(hid_scr[:, 0:_D2] * cat_row[:, 0:_D2],
                        axis=1, keepdims=True)
        h_col = h_col + jnp.sum(hid_scr[:, _D2:] * cat_row[:, _D2:],
                                axis=1, keepdims=True)
        x = _leaky(jnp.transpose(h_col) + hidb_ref[i:i + 1, :])
    out_ref[:] = x


def kernel(energy, word_h, e1, e2, sent_len, rel_embs, arc_W, arc_b,
           kv_W, kv_b, hid_W, hid_b):
    e2d = energy.reshape(_R, _LL)
    wr = word_h[0]
    wh_p = arc_W[:, :_D2]
    wrp_p = arc_W[:, _D2:_D2 + _EMB]
    wd_p = arc_W[:, _D2 + _EMB:]
    ab_col = arc_b.reshape(_D2, 1)

    at, bt, ct = pl.pallas_call(
        _phase0_body,
        out_shape=[
            jax.ShapeDtypeStruct((_D2, _L), jnp.float32),
            jax.ShapeDtypeStruct((_D2, _L), jnp.float32),
            jax.ShapeDtypeStruct((_D2, _R), jnp.float32),
        ],
    )(wr, wh_p, wd_p, wrp_p, rel_embs)

    at3 = jnp.pad(
        at.reshape(_D2, _NBLK, _IPB).transpose(1, 0, 2),
        ((0, 0), (0, 0), (0, _L - _IPB)))

    mbT, marg = pl.pallas_call(
        _phase1_body,
        grid=(_NBLK,),
        in_specs=[
            pl.BlockSpec((_R, _COLS), lambda b: (0, b)),
            pl.BlockSpec((1, _D2, _L), lambda b: (b, 0, 0)),
            pl.BlockSpec((_D2, _L), lambda b: (0, 0)),
            pl.BlockSpec((_D2, _R), lambda b: (0, 0)),
            pl.BlockSpec((_D2, 1), lambda b: (0, 0)),
        ],
        out_specs=[
            pl.BlockSpec((_D2, _COLS), lambda b: (0, b)),
            pl.BlockSpec((1, _COLS), lambda b: (0, b)),
        ],
        out_shape=[
            jax.ShapeDtypeStruct((_D2, _LL), jnp.float32),
            jax.ShapeDtypeStruct((1, _LL), jnp.float32),
        ],
    )(e2d, at3, bt, ct, ab_col)

    sl = jnp.asarray(sent_len, jnp.int32).reshape(1, 1)
    e12 = jnp.concatenate([e1, e2], axis=0).reshape(1, _D4)
    kvbt = kv_b.T

    out = pl.pallas_call(
        _phase2_body,
        in_specs=[
            _ANY_SPEC if idx in (4, 6) else pl.BlockSpec()
            for idx in range(8)
        ],
        out_shape=jax.ShapeDtypeStruct((1, _D4), jnp.float32),
        scratch_shapes=[
            pltpu.VMEM((_D4, _D4), jnp.float32),
            pltpu.VMEM((_D4, _D4), jnp.float32),
            pltpu.SemaphoreType.DMA,
            pltpu.SemaphoreType.DMA,
        ],
    )(mbT, marg, sl, e12, kv_W, kvbt, hid_W, hid_b)
    return out.reshape(_D4)


# speedup vs baseline: 4.8352x; 1.1959x over previous
import jax
import jax.numpy as jnp
from jax.experimental import pallas as pl
from jax.experimental.pallas import tpu as pltpu

_L = 128
_R = 40
_D2 = 512
_D4 = 1024
_EMB = 15
_LL = _L * _L
_NBLK = 8
_COLS = _LL // _NBLK
_IPB = _COLS // _L

_HIGH = jax.lax.Precision.HIGHEST
_HBM_SPEC = pl.BlockSpec(memory_space=pltpu.MemorySpace.HBM)


def _leaky(x):
    return jnp.where(x >= 0, x, 0.01 * x)


def _dotT(a, b, precision):
    return jax.lax.dot_general(
        a, b, (((1,), (1,)), ((), ())),
        precision=precision, preferred_element_type=jnp.float32)


def _phase0_body(wr_ref, wh_ref, wd_ref, wrp_ref, rel_ref,
                 at_ref, bt_ref, ct_ref):
    at_ref[:] = _dotT(wh_ref[:], wr_ref[:], _HIGH)
    bt_ref[:] = _dotT(wd_ref[:], wr_ref[:], _HIGH)
    ct_ref[:] = _dotT(wrp_ref[:], rel_ref[:], _HIGH)


def _main_body(e_ref, at3_ref, bt_ref, ct_ref, ab_ref, sl_ref, e12_ref,
               kvw_hbm, kvbt_ref, hidw_hbm, hidb_ref, out_ref,
               mb_scr, marg_scr, kv_scr, hid_scr, sem1, sem2):
    b = pl.program_id(0)

    @pl.when(b == 0)
    def _prefetch_hop0():
        pltpu.make_async_copy(kvw_hbm.at[0], kv_scr, sem1).start()
        pltpu.make_async_copy(hidw_hbm.at[0], hid_scr, sem2).start()

    @pl.when(b < _NBLK)
    def _build():
        e_blk = e_ref[:]
        relcT = jax.lax.dot_general(
            ct_ref[:], e_blk, (((1,), (0,)), ((), ())),
            preferred_element_type=jnp.float32)
        w_row = jnp.sum(e_blk, axis=0, keepdims=True)
        marg_scr[b] = w_row
        ab = ab_ref[:]
        bt = bt_ref[:]
        for t in range(_IPB):
            a_col = at3_ref[0, :, t:t + 1]
            sl_t = slice(t * _L, (t + 1) * _L)
            z = w_row[:, sl_t] * (a_col + bt) + relcT[:, sl_t] + ab
            mb_scr[b, :, sl_t] = _leaky(z)

    @pl.when(b == _NBLK)
    def _hops():
        sl = sl_ref[0, 0]
        t0 = jnp.float32(1e-06)
        t1 = jnp.float32(1e-06 * 0.1)
        t2 = jnp.float32(1e-06 * 0.01)
        margs, valids = [], []
        for c in range(_NBLK):
            margs.append(marg_scr[c])
            pidx = c * _COLS + jax.lax.broadcasted_iota(
                jnp.int32, (1, _COLS), 1)
            valids.append(((pidx // _L) < sl) & ((pidx % _L) < sl))
        c0 = sum(jnp.sum(((m > t0) & v).astype(jnp.float32))
                 for m, v in zip(margs, valids))
        c1 = sum(jnp.sum(((m > t1) & v).astype(jnp.float32))
                 for m, v in zip(margs, valids))
        thr = jnp.where(c0 > 0, t0, jnp.where(c1 > 0, t1, t2))
        stats = [(m > thr) & v for m, v in zip(margs, valids)]

        x = e12_ref[:]
        for i in range(3):
            if i > 0:
                pltpu.make_async_copy(kvw_hbm.at[i], kv_scr, sem1).start()
                pltpu.make_async_copy(hidw_hbm.at[i], hid_scr, sem2).start()
            pltpu.make_async_copy(kvw_hbm.at[i], kv_scr, sem1).wait()
            kv_col = jnp.concatenate(
                [jnp.sum(kv_scr[h * _D2:(h + 1) * _D2, :] * x,
                         axis=1, keepdims=True) for h in range(2)],
                axis=0) + kvbt_ref[:, i:i + 1]
            key_col = jnp.tanh(kv_col[:_D2])
            val_col = _leaky(kv_col[_D2:])
            logits = [jnp.sum(mb_scr[c] * key_col, axis=0, keepdims=True)
                      for c in range(_NBLK)]
            m = jnp.max(jnp.concatenate(
                [jnp.max(jnp.where(st, lg, jnp.float32(-1e30)),
                         axis=1, keepdims=True)
                 for st, lg in zip(stats, logits)], axis=0))
            exs = [jnp.where(st, jnp.exp(lg - m), jnp.float32(0.0))
                   for st, lg in zip(stats, logits)]
            s = sum(jnp.sum(ex) for ex in exs)
            mem_rep = sum(jnp.sum(mb_scr[c] * exs[c], axis=1, keepdims=True)
                          for c in range(_NBLK)) / s
            cat_row = jnp.transpose(
                jnp.concatenate([val_col, mem_rep], axis=0))
            pltpu.make_async_copy(hidw_hbm.at[i], hid_scr, sem2).wait()
            h_col = jnp.sum(hid_scr[:, 0:_D2] * cat_row[:, 0:_D2],
                            axis=1, keepdims=True)
            h_col = h_col + jnp.sum(hid_scr[:, _D2:] * cat_row[:, _D2:],
                                    axis=1, keepdims=True)
            x = _leaky(jnp.transpose(h_col) + hidb_ref[i:i + 1, :])
        out_ref[:] = x


def kernel(energy, word_h, e1, e2, sent_len, rel_embs, arc_W, arc_b,
           kv_W, kv_b, hid_W, hid_b):
    e2d = energy.reshape(_R, _LL)
    wr = word_h[0]
    wh_p = arc_W[:, :_D2]
    wrp_p = arc_W[:, _D2:_D2 + _EMB]
    wd_p = arc_W[:, _D2 + _EMB:]
    ab_col = arc_b.reshape(_D2, 1)

    at, bt, ct = pl.pallas_call(
        _phase0_body,
        out_shape=[
            jax.ShapeDtypeStruct((_D2, _L), jnp.float32),
            jax.ShapeDtypeStruct((_D2, _L), jnp.float32),
            jax.ShapeDtypeStruct((_D2, _R), jnp.float32),
        ],
    )(wr, wh_p, wd_p, wrp_p, rel_embs)

    at3 = jnp.pad(
        at.reshape(_D2, _NBLK, _IPB).transpose(1, 0, 2),
        ((0, 0), (0, 0), (0, _L - _IPB)))

    sl = jnp.asarray(sent_len, jnp.int32).reshape(1, 1)
    e12 = jnp.concatenate([e1, e2], axis=0).reshape(1, _D4)
    kvbt = kv_b.T

    last = _NBLK - 1
    out = pl.pallas_call(
        _main_body,
        grid=(_NBLK + 1,),
        in_specs=[
            pl.BlockSpec((_R, _COLS), lambda b: (0, jnp.minimum(b, last))),
            pl.BlockSpec((1, _D2, _L), lambda b: (jnp.minimum(b, last), 0, 0)),
            pl.BlockSpec((_D2, _L), lambda b: (0, 0)),
            pl.BlockSpec((_D2, _R), lambda b: (0, 0)),
            pl.BlockSpec((_D2, 1), lambda b: (0, 0)),
            pl.BlockSpec((1, 1), lambda b: (0, 0)),
            pl.BlockSpec((1, _D4), lambda b: (0, 0)),
            _HBM_SPEC,
            pl.BlockSpec((_D4, 3), lambda b: (0, 0)),
            _HBM_SPEC,
            pl.BlockSpec((3, _D4), lambda b: (0, 0)),
        ],
        out_specs=pl.BlockSpec((1, _D4), lambda b: (0, 0)),
        out_shape=jax.ShapeDtypeStruct((1, _D4), jnp.float32),
        scratch_shapes=[
            pltpu.VMEM((_NBLK, _D2, _COLS), jnp.float32),
            pltpu.VMEM((_NBLK, 1, _COLS), jnp.float32),
            pltpu.VMEM((_D4, _D4), jnp.float32),
            pltpu.VMEM((_D4, _D4), jnp.float32),
            pltpu.SemaphoreType.DMA,
            pltpu.SemaphoreType.DMA,
        ],
    )(e2d, at3, bt, ct, ab_col, sl, e12, kv_W, kvbt, hid_W, hid_b)
    return out.reshape(_D4)


# speedup vs baseline: 6.8312x; 1.4128x over previous
import jax
import jax.numpy as jnp
from jax.experimental import pallas as pl
from jax.experimental.pallas import tpu as pltpu

_L = 128
_R = 40
_D2 = 512
_D4 = 1024
_EMB = 15
_LL = _L * _L
_NBLK = 8
_COLS = _LL // _NBLK
_IPB = _COLS // _L

_HIGH = jax.lax.Precision.HIGHEST
_HBM_SPEC = pl.BlockSpec(memory_space=pltpu.MemorySpace.HBM)


def _leaky(x):
    return jnp.where(x >= 0, x, 0.01 * x)


def _dotT(a, b, precision):
    return jax.lax.dot_general(
        a, b, (((1,), (1,)), ((), ())),
        precision=precision, preferred_element_type=jnp.float32)


def _main_body(e_ref, wr3_ref, arc_ref, rel_ref,
               ab_ref, sl_ref, e1_ref, e2_ref,
               kvw_hbm, kvb_ref, hidw_hbm, hidb_ref, out_ref,
               mb_scr, marg_scr, at3_scr, bt_scr, ct_scr, k0_scr, v0_scr,
               lg0_scr, kv_scr, hid_scr, sem1, sem2, sem3):
    b = pl.program_id(0)

    @pl.when(b == 0)
    def _init():
        pltpu.make_async_copy(kvw_hbm.at[0], kv_scr.at[0], sem1).start()
        pltpu.make_async_copy(hidw_hbm.at[0], hid_scr, sem2).start()
        pltpu.make_async_copy(kvw_hbm.at[1], kv_scr.at[1], sem3).start()
        wr = wr3_ref[0]
        at = _dotT(arc_ref[:, 0:_D2], wr, _HIGH)
        bt_scr[:] = _dotT(arc_ref[:, _D2 + _EMB:], wr, _HIGH)
        ct_scr[:] = _dotT(arc_ref[:, _D2:_D2 + _EMB], rel_ref[:], _HIGH)
        pad = jnp.zeros((_D2, _L - _IPB), jnp.float32)
        for bb in range(_NBLK):
            at3_scr[bb] = jnp.concatenate(
                [at[:, bb * _IPB:(bb + 1) * _IPB], pad], axis=1)
        x0 = jnp.concatenate([e1_ref[:], e2_ref[:]], axis=1)
        pltpu.make_async_copy(kvw_hbm.at[0], kv_scr.at[0], sem1).wait()
        kv_col = jnp.concatenate(
            [jnp.sum(kv_scr[0, h * _D2:(h + 1) * _D2, :] * x0,
                     axis=1, keepdims=True) for h in range(2)],
            axis=0) + jnp.transpose(kvb_ref[0:1, :])
        k0_scr[:] = jnp.tanh(kv_col[:_D2])
        v0_scr[:] = _leaky(kv_col[_D2:])

    @pl.when(b == 1)
    def _refill_kv0():
        pltpu.make_async_copy(kvw_hbm.at[2], kv_scr.at[0], sem1).start()

    @pl.when(b < _NBLK)
    def _build():
        e_blk = e_ref[:]
        relcT = jax.lax.dot_general(
            ct_scr[:], e_blk, (((1,), (0,)), ((), ())),
            preferred_element_type=jnp.float32)
        w_row = jnp.sum(e_blk, axis=0, keepdims=True)
        marg_scr[b] = w_row
        ab = jnp.transpose(ab_ref[:])
        bt = bt_scr[:]
        k0 = k0_scr[:]
        for t in range(_IPB):
            a_col = at3_scr[b, :, t:t + 1]
            sl_t = slice(t * _L, (t + 1) * _L)
            z = w_row[:, sl_t] * (a_col + bt) + relcT[:, sl_t] + ab
            zl = _leaky(z)
            mb_scr[b, :, sl_t] = zl
            lg0_scr[b, :, sl_t] = jnp.sum(zl * k0, axis=0, keepdims=True)

    @pl.when(b == _NBLK)
    def _hops():
        sl = sl_ref[0, 0]
        t0 = jnp.float32(1e-06)
        t1 = jnp.float32(1e-06 * 0.1)
        t2 = jnp.float32(1e-06 * 0.01)
        margs, valids = [], []
        for c in range(_NBLK):
            margs.append(marg_scr[c])
            pidx = c * _COLS + jax.lax.broadcasted_iota(
                jnp.int32, (1, _COLS), 1)
            valids.append(((pidx // _L) < sl) & ((pidx % _L) < sl))
        c0 = sum(jnp.sum(((m > t0) & v).astype(jnp.float32))
                 for m, v in zip(margs, valids))
        c1 = sum(jnp.sum(((m > t1) & v).astype(jnp.float32))
                 for m, v in zip(margs, valids))
        thr = jnp.where(c0 > 0, t0, jnp.where(c1 > 0, t1, t2))
        stats = [(m > thr) & v for m, v in zip(margs, valids)]

        x = jnp.concatenate([e1_ref[:], e2_ref[:]], axis=1)
        for i in range(3):
            buf = i % 2
            if i == 0:
                val_col = v0_scr[:]
                logits = [lg0_scr[c] for c in range(_NBLK)]
            else:
                ksem = sem1 if buf == 0 else sem3
                pltpu.make_async_copy(kvw_hbm.at[i], kv_scr.at[buf],
                                      ksem).wait()
                kv_col = jnp.concatenate(
                    [jnp.sum(kv_scr[buf, h * _D2:(h + 1) * _D2, :] * x,
                             axis=1, keepdims=True) for h in range(2)],
                    axis=0) + jnp.transpose(kvb_ref[i:i + 1, :])
                key_col = jnp.tanh(kv_col[:_D2])
                val_col = _leaky(kv_col[_D2:])
                logits = [jnp.sum(mb_scr[c] * key_col, axis=0, keepdims=True)
                          for c in range(_NBLK)]
            m = jnp.max(jnp.concatenate(
                [jnp.max(jnp.where(st, lg, jnp.float32(-1e30)),
                         axis=1, keepdims=True)
                 for st, lg in zip(stats, logits)], axis=0))
            exs = [jnp.where(st, jnp.exp(lg - m), jnp.float32(0.0))
                   for st, lg in zip(stats, logits)]
            s = sum(jnp.sum(ex) for ex in exs)
            mem_rep = sum(jnp.sum(mb_scr[c] * exs[c], axis=1, keepdims=True)
                          for c in range(_NBLK)) / s
            cat_row = jnp.transpose(
                jnp.concatenate([val_col, mem_rep], axis=0))
            pltpu.make_async_copy(hidw_hbm.at[i], hid_scr, sem2).wait()
            h_col = jnp.sum(hid_scr[:, 0:_D2] * cat_row[:, 0:_D2],
                            axis=1, keepdims=True)
            h_col = h_col + jnp.sum(hid_scr[:, _D2:] * cat_row[:, _D2:],
                                    axis=1, keepdims=True)
            x = _leaky(jnp.transpose(h_col) + hidb_ref[i:i + 1, :])
            if i < 2:
                pltpu.make_async_copy(hidw_hbm.at[i + 1], hid_scr,
                                      sem2).start()
        out_ref[:] = x


def kernel(energy, word_h, e1, e2, sent_len, rel_embs, arc_W, arc_b,
           kv_W, kv_b, hid_W, hid_b):
    e2d = energy.reshape(_R, _LL)
    sl = jnp.asarray(sent_len, jnp.int32).reshape(1, 1)
    e1r = e1.reshape(1, _D2)
    e2r = e2.reshape(1, _D2)
    abr = arc_b.reshape(1, _D2)

    last = _NBLK - 1
    out = pl.pallas_call(
        _main_body,
        grid=(_NBLK + 1,),
        in_specs=[
            pl.BlockSpec((_R, _COLS), lambda b: (0, jnp.minimum(b, last))),
            pl.BlockSpec((1, _L, _D2), lambda b: (0, 0, 0)),
            pl.BlockSpec((_D2, _D2 * 2 + _EMB), lambda b: (0, 0)),
            pl.BlockSpec((_R, _EMB), lambda b: (0, 0)),
            pl.BlockSpec((1, _D2), lambda b: (0, 0)),
            pl.BlockSpec((1, 1), lambda b: (0, 0)),
            pl.BlockSpec((1, _D2), lambda b: (0, 0)),
            pl.BlockSpec((1, _D2), lambda b: (0, 0)),
            _HBM_SPEC,
            pl.BlockSpec((3, _D4), lambda b: (0, 0)),
            _HBM_SPEC,
            pl.BlockSpec((3, _D4), lambda b: (0, 0)),
        ],
        out_specs=pl.BlockSpec((1, _D4), lambda b: (0, 0)),
        out_shape=jax.ShapeDtypeStruct((1, _D4), jnp.float32),
        scratch_shapes=[
            pltpu.VMEM((_NBLK, _D2, _COLS), jnp.float32),
            pltpu.VMEM((_NBLK, 1, _COLS), jnp.float32),
            pltpu.VMEM((_NBLK, _D2, _L), jnp.float32),
            pltpu.VMEM((_D2, _L), jnp.float32),
            pltpu.VMEM((_D2, _R), jnp.float32),
            pltpu.VMEM((_D2, 1), jnp.float32),
            pltpu.VMEM((_D2, 1), jnp.float32),
            pltpu.VMEM((_NBLK, 1, _COLS), jnp.float32),
            pltpu.VMEM((2, _D4, _D4), jnp.float32),
            pltpu.VMEM((_D4, _D4), jnp.float32),
            pltpu.SemaphoreType.DMA,
            pltpu.SemaphoreType.DMA,
            pltpu.SemaphoreType.DMA,
        ],
    )(e2d, word_h, arc_W, rel_embs, abr, sl, e1r, e2r,
      kv_W, kv_b, hid_W, hid_b)
    return out.reshape(_D4)
